# baseline (speedup 1.0000x reference)
import functools

import jax
import jax.numpy as jnp
from jax import lax
from jax.experimental import pallas as pl
from jax.experimental.pallas import tpu as pltpu
from jax.experimental.pallas import tpu_sc as plsc

N = 10000
D_IN = 128
D_HID = 16
D_OUT = 7

NC = 2
NS = 16
NW = NC * NS

NP = 10240
EB = 128
E_ROWS = 2500
RWB = E_ROWS // NW
RW_MAX = RWB + 1
ROWS_PER_TILE = NP // NS
TROWS = N // NS
SROWS = 313
NO = NW * SROWS

_f32 = jnp.float32

_mesh = plsc.VectorSubcoreMesh(
    core_axis_name="c", subcore_axis_name="s",
    num_cores=NC, num_subcores=NS)

_sc_params = pltpu.CompilerParams(use_tc_tiling_on_sc=False,
                                  needs_layout_passes=False)


def _stage_indices(ei_hbm, idx_s, idx_d, wid):
  extra = wid < (E_ROWS - NW * RWB)
  nrows = RWB + extra.astype(jnp.int32)
  pltpu.sync_copy(ei_hbm.at[0, pl.ds(wid * RWB, RWB)],
                  idx_s.at[pl.ds(0, RWB)])
  pltpu.sync_copy(ei_hbm.at[1, pl.ds(wid * RWB, RWB)],
                  idx_d.at[pl.ds(0, RWB)])

  @pl.when(extra)
  def _():
    pltpu.sync_copy(ei_hbm.at[0, NW * RWB + wid], idx_s.at[RWB])
    pltpu.sync_copy(ei_hbm.at[1, NW * RWB + wid], idx_d.at[RWB])

  return nrows


def _edge_pipeline(nrows, idx_s, idx_d, tbls, acc, cnta, ones_v, rows,
                   semg, sems, semc, with_counts):
  def fire_gather(j, slot):
    pltpu.async_copy(tbls.at[idx_s.at[j]], rows.at[slot], semg)

  def wait_gather():
    pltpu.make_async_copy(tbls.at[idx_s.at[0]], rows.at[0], semg).wait()

  def fire_scatter(j, slot):
    pltpu.async_copy(rows.at[slot], acc.at[idx_d.at[j]], sems, add=True)
    if with_counts:
      pltpu.async_copy(ones_v, cnta.at[idx_d.at[j]], semc, add=True)

  def wait_scatter():
    pltpu.make_async_copy(rows.at[0], acc.at[idx_d.at[0]], sems).wait()
    if with_counts:
      pltpu.make_async_copy(ones_v, cnta.at[idx_d.at[0]], semc).wait()

  fire_gather(0, 0)

  def step(j, _):
    slot = lax.rem(j, 2)
    wait_gather()

    @pl.when(j >= 1)
    def _():
      wait_scatter()

    @pl.when(j + 1 < nrows)
    def _():
      fire_gather(j + 1, 1 - slot)

    fire_scatter(j, slot)
    return 0

  lax.fori_loop(0, nrows, step, 0)
  wait_scatter()


_b_scratch = [
    pltpu.VMEM((RW_MAX, EB), jnp.int32),
    pltpu.VMEM((RW_MAX, EB), jnp.int32),
    pltpu.VMEM((2, EB, D_HID), _f32),
    pltpu.VMEM((EB, D_HID), _f32),
    pltpu.VMEM_SHARED((NP, D_HID), _f32),
    pltpu.VMEM_SHARED((NP, D_HID), _f32),
    pltpu.VMEM_SHARED((N, D_HID), _f32),
    pltpu.SemaphoreType.DMA,
    pltpu.SemaphoreType.DMA,
    pltpu.SemaphoreType.DMA,
]


@functools.partial(
    pl.kernel,
    out_type=[jax.ShapeDtypeStruct((NC, NP, D_HID), _f32),
              jax.ShapeDtypeStruct((NC, NP, D_HID), _f32)],
    mesh=_mesh, scratch_types=_b_scratch, compiler_params=_sc_params)
def _sc_layer1(ei_hbm, y1_hbm, z2_hbm, ones_hbm, agg_hbm, cnt_hbm,
               idx_s, idx_d, rows, ones_v, acc, cnta, tbls,
               semg, sems, semc):
  cid = lax.axis_index("c")
  sid = lax.axis_index("s")
  wid = cid * NS + sid
  r0 = sid * ROWS_PER_TILE
  t0 = sid * TROWS

  pltpu.sync_copy(y1_hbm.at[pl.ds(t0, TROWS), pl.ds(0, D_HID)],
                  tbls.at[pl.ds(t0, TROWS)])
  pltpu.sync_copy(z2_hbm.at[pl.ds(r0, ROWS_PER_TILE)],
                  acc.at[pl.ds(r0, ROWS_PER_TILE)])
  pltpu.sync_copy(z2_hbm.at[pl.ds(r0, ROWS_PER_TILE)],
                  cnta.at[pl.ds(r0, ROWS_PER_TILE)])
  pltpu.sync_copy(ones_hbm, ones_v)
  nrows = _stage_indices(ei_hbm, idx_s, idx_d, wid)
  plsc.subcore_barrier()

  _edge_pipeline(nrows, idx_s, idx_d, tbls, acc, cnta, ones_v, rows,
                 semg, sems, semc, True)
  plsc.subcore_barrier()

  pltpu.sync_copy(acc.at[pl.ds(r0, ROWS_PER_TILE)],
                  agg_hbm.at[cid, pl.ds(r0, ROWS_PER_TILE)])
  pltpu.sync_copy(cnta.at[pl.ds(r0, ROWS_PER_TILE)],
                  cnt_hbm.at[cid, pl.ds(r0, ROWS_PER_TILE)])


_c_scratch = [
    pltpu.VMEM((RW_MAX, EB), jnp.int32),
    pltpu.VMEM((RW_MAX, EB), jnp.int32),
    pltpu.VMEM((2, EB, D_HID), _f32),
    pltpu.VMEM((TROWS, D_HID), _f32),
    pltpu.VMEM((TROWS, D_HID), _f32),
    pltpu.VMEM((TROWS, D_HID), _f32),
    pltpu.VMEM((TROWS, D_HID), _f32),
    pltpu.VMEM((TROWS, D_HID), _f32),
    pltpu.VMEM((TROWS, 2 * D_HID), _f32),
    pltpu.VMEM((2 * D_HID * D_HID,), _f32),
    pltpu.VMEM((D_HID,), _f32),
    pltpu.VMEM_SHARED((NP, D_HID), _f32),
    pltpu.VMEM_SHARED((N, D_HID), _f32),
    pltpu.SemaphoreType.DMA,
    pltpu.SemaphoreType.DMA,
    pltpu.SemaphoreType.DMA,
]


@functools.partial(
    pl.kernel,
    out_type=[jax.ShapeDtypeStruct((NC, NP, D_HID), _f32),
              jax.ShapeDtypeStruct((NP, 2 * D_HID), _f32)],
    mesh=_mesh, scratch_types=_c_scratch, compiler_params=_sc_params)
def _sc_mid(ei_hbm, agg1_hbm, cnt_hbm, y1_hbm, w2_hbm, b1_hbm, z2_hbm,
            agg_hbm, r2iv_hbm,
            idx_s, idx_d, rows, a0v, a1v, c0v, c1v, r1v, r2ivv,
            w2v, b1v, acc, tbls, semg, sems, semc):
  cid = lax.axis_index("c")
  sid = lax.axis_index("s")
  wid = cid * NS + sid
  r0 = sid * ROWS_PER_TILE
  t0 = sid * TROWS

  pltpu.sync_copy(agg1_hbm.at[0, pl.ds(t0, TROWS)], a0v)
  pltpu.sync_copy(agg1_hbm.at[1, pl.ds(t0, TROWS)], a1v)
  pltpu.sync_copy(cnt_hbm.at[0, pl.ds(t0, TROWS)], c0v)
  pltpu.sync_copy(cnt_hbm.at[1, pl.ds(t0, TROWS)], c1v)
  pltpu.sync_copy(y1_hbm.at[pl.ds(t0, TROWS), pl.ds(D_HID, D_HID)], r1v)
  pltpu.sync_copy(w2_hbm, w2v)
  pltpu.sync_copy(b1_hbm, b1v)
  pltpu.sync_copy(z2_hbm.at[pl.ds(r0, ROWS_PER_TILE)],
                  acc.at[pl.ds(r0, ROWS_PER_TILE)])
  nrows = _stage_indices(ei_hbm, idx_s, idx_d, wid)

  b1row = b1v[...]
  wl2 = [w2v[pl.ds(k * D_HID, D_HID)] for k in range(D_HID)]
  wr2 = [w2v[pl.ds((D_HID + k) * D_HID, D_HID)] for k in range(D_HID)]

  def dense_row(r, _):
    cnt = c0v[r, :] + c1v[r, :]
    iv = 1.0 / jnp.maximum(cnt, 1.0)
    h = jnp.maximum((a0v[r, :] + a1v[r, :]) * iv + b1row + r1v[r, :], 0.0)
    a0v[r, :] = h
    r2ivv[r, pl.ds(D_HID, D_HID)] = iv
    return 0

  lax.fori_loop(0, TROWS, dense_row, 0)

  def proj_row(r, _):
    hrow = a0v[r, :]
    p2 = jnp.zeros((D_HID,), _f32)
    r2 = jnp.zeros((D_HID,), _f32)
    for k in range(D_HID):
      s = hrow[k]
      p2 = p2 + s * wl2[k]
      r2 = r2 + s * wr2[k]
    c0v[r, :] = p2
    r2ivv[r, pl.ds(0, D_HID)] = r2
    return 0

  lax.fori_loop(0, TROWS, proj_row, 0)

  pltpu.sync_copy(c0v, tbls.at[pl.ds(t0, TROWS)])

  @pl.when(cid == 0)
  def _():
    pltpu.sync_copy(r2ivv, r2iv_hbm.at[pl.ds(t0, TROWS)])

  plsc.subcore_barrier()

  _edge_pipeline(nrows, idx_s, idx_d, tbls, acc, None, None, rows,
                 semg, sems, semc, False)
  plsc.subcore_barrier()

  pltpu.sync_copy(acc.at[pl.ds(r0, ROWS_PER_TILE)],
                  agg_hbm.at[cid, pl.ds(r0, ROWS_PER_TILE)])


_e_scratch = [
    pltpu.VMEM((SROWS, D_HID), _f32),
    pltpu.VMEM((SROWS, D_HID), _f32),
    pltpu.VMEM((SROWS, 2 * D_HID), _f32),
    pltpu.VMEM((D_HID,), _f32),
    pltpu.VMEM((SROWS, D_HID), _f32),
]

_LN2 = 0.6931471805599453


@functools.partial(
    pl.kernel,
    out_type=[jax.ShapeDtypeStruct((NO, D_HID), _f32)],
    mesh=_mesh, scratch_types=_e_scratch, compiler_params=_sc_params)
def _sc_out(agg_hbm, r2iv_hbm, b2_hbm, o_hbm, a0v, a1v, r2ivv, b2v, ov):
  cid = lax.axis_index("c")
  sid = lax.axis_index("s")
  wid = cid * NS + sid
  t0 = wid * SROWS

  pltpu.sync_copy(agg_hbm.at[0, pl.ds(t0, SROWS)], a0v)
  pltpu.sync_copy(agg_hbm.at[1, pl.ds(t0, SROWS)], a1v)
  pltpu.sync_copy(r2iv_hbm.at[pl.ds(t0, SROWS)], r2ivv)
  pltpu.sync_copy(b2_hbm, b2v)

  b2row = b2v[...]
  mask = lax.iota(jnp.int32, D_HID) < D_OUT

  def row(r, _):
    iv = r2ivv[r, pl.ds(D_HID, D_HID)]
    z = (a0v[r, :] + a1v[r, :]) * iv + b2row + r2ivv[r, pl.ds(0, D_HID)]
    m = jnp.max(jnp.where(mask, z, -3.0e38))
    e = jnp.where(mask, jnp.exp(z - m), 0.0)
    s = jnp.sum(e)
    sv = jnp.full((D_HID,), s, _f32)
    bits = plsc.bitcast(sv, jnp.int32)
    ex = ((bits >> 23) & 0xFF) - 127
    mant = plsc.bitcast((bits & 0x7FFFFF) | 0x3F800000, _f32)
    w = (mant - 1.0) / (mant + 1.0)
    w2 = w * w
    poly = 1.0 + w2 * (1.0 / 3.0 + w2 * (1.0 / 5.0 + w2 * (1.0 / 7.0
                + w2 * (1.0 / 9.0))))
    lse = ex.astype(_f32) * _LN2 + 2.0 * w * poly
    ov[r, :] = z - m - lse
    return 0

  lax.fori_loop(0, SROWS, row, 0)
  pltpu.sync_copy(ov, o_hbm.at[pl.ds(t0, SROWS)])


def _tc_proj_kernel(x_ref, w_ref, y_ref):
  y_ref[...] = jnp.dot(x_ref[...], w_ref[...],
                       preferred_element_type=_f32)


_BN = 1000


def kernel(x, edge_index, W_l1, b1, W_r1, W_l2, b2, W_r2):
  ei3 = edge_index.reshape(2, E_ROWS, EB)
  z2 = jnp.zeros((NP, D_HID), _f32)
  ones = jnp.ones((EB, D_HID), _f32)

  w1 = jnp.concatenate(
      [W_l1, W_r1, jnp.zeros((D_IN, D_IN - 2 * D_HID), _f32)], axis=1)
  y1 = pl.pallas_call(
      _tc_proj_kernel,
      grid=(N // _BN,),
      in_specs=[
          pl.BlockSpec((_BN, D_IN), lambda i: (i, 0)),
          pl.BlockSpec((D_IN, D_IN), lambda i: (0, 0)),
      ],
      out_specs=pl.BlockSpec((_BN, D_IN), lambda i: (i, 0)),
      out_shape=jax.ShapeDtypeStruct((N, D_IN), _f32),
  )(x, w1)

  agg1, cnt1 = _sc_layer1(ei3, y1, z2, ones)

  w2 = jnp.concatenate([
      jnp.pad(W_l2, ((0, 0), (0, D_HID - D_OUT))).reshape(-1),
      jnp.pad(W_r2, ((0, 0), (0, D_HID - D_OUT))).reshape(-1),
  ])
  agg2, r2iv = _sc_mid(ei3, agg1, cnt1, y1, w2, b1, z2)

  b2p = jnp.pad(b2, (0, D_HID - D_OUT))
  out16 = _sc_out(agg2, r2iv, b2p)
  if isinstance(out16, (list, tuple)):
    out16 = out16[0]
  return out16[:N, :D_OUT]
